# baseline (speedup 1.0000x reference)
import functools

import jax
import jax.numpy as jnp
from jax import lax
from jax.experimental import pallas as pl
from jax.experimental.pallas import tpu as pltpu
from jax.experimental.pallas import tpu_sc as plsc

_NUM_CORES = 2
_NUM_SUBCORES = 16
_NUM_WORKERS = _NUM_CORES * _NUM_SUBCORES


def _sc_kv_update(pos, k_val2, v_val2, k_cache2, v_cache2, *, n_slabs, S, Q, D):
    slabs_per = n_slabs // _NUM_WORKERS
    mesh = plsc.VectorSubcoreMesh(
        core_axis_name="c", subcore_axis_name="s",
        num_cores=_NUM_CORES, num_subcores=_NUM_SUBCORES)

    @functools.partial(
        pl.kernel,
        out_type=(
            jax.ShapeDtypeStruct((n_slabs * S, D), jnp.float32),
            jax.ShapeDtypeStruct((n_slabs * S, D), jnp.float32),
        ),
        mesh=mesh,
        scratch_types=[
            pltpu.VMEM((Q,), jnp.int32),
            pltpu.VMEM((slabs_per * Q, D), jnp.float32),
            pltpu.VMEM((slabs_per * Q, D), jnp.float32),
            pltpu.SemaphoreType.DMA,
            pltpu.SemaphoreType.DMA,
            pltpu.SemaphoreType.DMA,
        ],
    )
    def body(pos_hbm, kval_hbm, vval_hbm, kc_hbm, vc_hbm, kout_hbm, vout_hbm,
             pos_v, kv_v, vv_v, sem_copy, sem_val, sem_sc):
        wid = lax.axis_index("s") * _NUM_CORES + lax.axis_index("c")
        base = wid * slabs_per

        row0 = base * S
        nrows = slabs_per * S
        ck = pltpu.make_async_copy(
            kc_hbm.at[pl.ds(row0, nrows)], kout_hbm.at[pl.ds(row0, nrows)],
            sem_copy)
        cv = pltpu.make_async_copy(
            vc_hbm.at[pl.ds(row0, nrows)], vout_hbm.at[pl.ds(row0, nrows)],
            sem_copy)
        ck.start()
        cv.start()

        vrow0 = base * Q
        nvrows = slabs_per * Q
        lk = pltpu.make_async_copy(
            kval_hbm.at[pl.ds(vrow0, nvrows)], kv_v, sem_val)
        lv = pltpu.make_async_copy(
            vval_hbm.at[pl.ds(vrow0, nvrows)], vv_v, sem_val)
        lk.start()
        lv.start()
        pltpu.sync_copy(pos_hbm, pos_v)
        lk.wait()
        lv.wait()
        ck.wait()
        cv.wait()

        pos_vec = pos_v[...]
        scs = []
        for j in range(slabs_per):
            idx = pos_vec + (base + j) * S
            src_k = kv_v.at[pl.ds(j * Q, Q)]
            src_v = vv_v.at[pl.ds(j * Q, Q)]
            sk = pltpu.make_async_copy(src_k, kout_hbm.at[idx], sem_sc)
            sv = pltpu.make_async_copy(src_v, vout_hbm.at[idx], sem_sc)
            sk.start()
            sv.start()
            scs.append(sk)
            scs.append(sv)
        for c in scs:
            c.wait()

    return body(pos, k_val2, v_val2, k_cache2, v_cache2)


def kernel(input_pos, k_val, v_val, k_cache, v_cache):
    B, H, Q, D = k_val.shape
    S = k_cache.shape[2]
    n_slabs = B * H
    pos = input_pos.astype(jnp.int32)
    k_out2, v_out2 = _sc_kv_update(
        pos,
        k_val.reshape(n_slabs * Q, D),
        v_val.reshape(n_slabs * Q, D),
        k_cache.reshape(n_slabs * S, D),
        v_cache.reshape(n_slabs * S, D),
        n_slabs=n_slabs, S=S, Q=Q, D=D)
    return (k_out2.reshape(B, H, S, D), v_out2.reshape(B, H, S, D))
